# baseline (speedup 1.0000x reference)
import functools

import jax
import jax.numpy as jnp
from jax import lax
from jax.experimental import pallas as pl
from jax.experimental.pallas import tpu as pltpu
from jax.experimental.pallas import tpu_sc as plsc

_N = 10000
_E = 320000
_B = 16
_EB = 2000
_GRID = _E // _EB
_CH = 80


def _sc_gather(src, dst, u, pw):
    info = plsc.get_sparse_core_info()
    nc, ns = info.num_cores, info.num_subcores
    nw = nc * ns
    per = _E // nw
    nch = per // _CH

    mesh = plsc.VectorSubcoreMesh(core_axis_name="c", subcore_axis_name="s")

    @functools.partial(
        pl.kernel, mesh=mesh,
        out_type=[jax.ShapeDtypeStruct((_E, 64), jnp.float32),
                  jax.ShapeDtypeStruct((_E, 64), jnp.float32)],
        scratch_types=[pltpu.VMEM((_CH,), jnp.int32),
                       pltpu.VMEM((_CH, 64), jnp.float32),
                       pltpu.SemaphoreType.DMA],
        compiler_params=pltpu.CompilerParams(use_tc_tiling_on_sc=False),
    )
    def k(src_hbm, dst_hbm, u_hbm, pw_hbm, gu_hbm, gp_hbm, idx_v, rows_v, sem):
        wid = lax.axis_index("s") * nc + lax.axis_index("c")
        def body(i, carry):
            base = wid * per + i * _CH
            pltpu.sync_copy(src_hbm.at[pl.ds(base, _CH)], idx_v)
            pltpu.async_copy(u_hbm.at[idx_v], rows_v, sem).wait()
            pltpu.sync_copy(rows_v, gu_hbm.at[pl.ds(base, _CH)])
            pltpu.sync_copy(dst_hbm.at[pl.ds(base, _CH)], idx_v)
            pltpu.async_copy(pw_hbm.at[idx_v], rows_v, sem).wait()
            pltpu.sync_copy(rows_v, gp_hbm.at[pl.ds(base, _CH)])
            return carry
        lax.fori_loop(0, nch, body, 0)

    return k(src, dst, u, pw)


def _stage_nodes(x, pos8, w1x, w1p8, b1):
    def body(xr, pr, wxr, wpr, br, ur, pwr):
        pwv = lax.dot_general(pr[...], wpr[...], (((1,), (1,)), ((), ())),
                              preferred_element_type=jnp.float32)
        xw = lax.dot_general(xr[...], wxr[...], (((1,), (1,)), ((), ())),
                             preferred_element_type=jnp.float32)
        pwr[...] = pwv
        ur[...] = xw + pwv + br[...]

    return pl.pallas_call(
        body,
        out_shape=[jax.ShapeDtypeStruct((_N, 64), jnp.float32),
                   jax.ShapeDtypeStruct((_N, 64), jnp.float32)],
    )(x, pos8, w1x, w1p8, b1)


def _stage_edge1(gu, gp):
    def body(gur, gpr, h1r, s1r, q1r):
        h = jnp.maximum(gur[...] - gpr[...], 0.0)
        h1r[...] = h
        @pl.when(pl.program_id(0) == 0)
        def _():
            s1r[...] = jnp.zeros((1, 64), jnp.float32)
            q1r[...] = jnp.zeros((1, 64), jnp.float32)
        s1r[...] += jnp.sum(h, axis=0, keepdims=True)
        q1r[...] += jnp.sum(h * h, axis=0, keepdims=True)

    return pl.pallas_call(
        body,
        grid=(_GRID,),
        in_specs=[pl.BlockSpec((_EB, 64), lambda i: (i, 0)),
                  pl.BlockSpec((_EB, 64), lambda i: (i, 0))],
        out_specs=[pl.BlockSpec((_EB, 64), lambda i: (i, 0)),
                   pl.BlockSpec((1, 64), lambda i: (0, 0)),
                   pl.BlockSpec((1, 64), lambda i: (0, 0))],
        out_shape=[jax.ShapeDtypeStruct((_E, 64), jnp.float32),
                   jax.ShapeDtypeStruct((1, 64), jnp.float32),
                   jax.ShapeDtypeStruct((1, 64), jnp.float32)],
    )(gu, gp)


def _stage_edge2(h1, s1, q1, g1, be1, w2, b2):
    def body(h1r, s1r, q1r, g1r, be1r, w2r, b2r, h2r, s2r, q2r):
        mu = s1r[...] / _E
        var = jnp.maximum(q1r[...] / _E - mu * mu, 0.0)
        rs = lax.rsqrt(var + 1e-5)
        h1n = g1r[...] * (h1r[...] - mu) * rs + be1r[...]
        h2 = jnp.maximum(
            lax.dot_general(h1n, w2r[...], (((1,), (1,)), ((), ())),
                            preferred_element_type=jnp.float32) + b2r[...], 0.0)
        h2r[...] = h2
        @pl.when(pl.program_id(0) == 0)
        def _():
            s2r[...] = jnp.zeros((1, 128), jnp.float32)
            q2r[...] = jnp.zeros((1, 128), jnp.float32)
        s2r[...] += jnp.sum(h2, axis=0, keepdims=True)
        q2r[...] += jnp.sum(h2 * h2, axis=0, keepdims=True)

    return pl.pallas_call(
        body,
        grid=(_GRID,),
        in_specs=[pl.BlockSpec((_EB, 64), lambda i: (i, 0)),
                  pl.BlockSpec((1, 64), lambda i: (0, 0)),
                  pl.BlockSpec((1, 64), lambda i: (0, 0)),
                  pl.BlockSpec((1, 64), lambda i: (0, 0)),
                  pl.BlockSpec((1, 64), lambda i: (0, 0)),
                  pl.BlockSpec((128, 64), lambda i: (0, 0)),
                  pl.BlockSpec((1, 128), lambda i: (0, 0))],
        out_specs=[pl.BlockSpec((_EB, 128), lambda i: (i, 0)),
                   pl.BlockSpec((1, 128), lambda i: (0, 0)),
                   pl.BlockSpec((1, 128), lambda i: (0, 0))],
        out_shape=[jax.ShapeDtypeStruct((_E, 128), jnp.float32),
                   jax.ShapeDtypeStruct((1, 128), jnp.float32),
                   jax.ShapeDtypeStruct((1, 128), jnp.float32)],
    )(h1, s1, q1, g1, be1, w2, b2)


def _stage_scatter(dst2, h2):
    def body(dstr, h2r, aggr):
        @pl.when(pl.program_id(0) == 0)
        def _():
            aggr[...] = jnp.full((_N, 128), -jnp.inf, jnp.float32)
        def loop(e, carry):
            d = dstr[e // 250, e % 250]
            row = h2r[pl.ds(e, 1), :]
            aggr[pl.ds(d, 1), :] = jnp.maximum(aggr[pl.ds(d, 1), :], row)
            return carry
        lax.fori_loop(0, _EB, loop, 0)

    return pl.pallas_call(
        body,
        grid=(_GRID,),
        in_specs=[pl.BlockSpec((8, 250), lambda i: (i, 0),
                               memory_space=pltpu.SMEM),
                  pl.BlockSpec((_EB, 128), lambda i: (i, 0))],
        out_specs=pl.BlockSpec((_N, 128), lambda i: (0, 0)),
        out_shape=jax.ShapeDtypeStruct((_N, 128), jnp.float32),
    )(dst2, h2)


def _stage_head(agg, s2, q2, g2, be2, pos8, batch2d,
                w3a, w3p8, b3, g3, be3, w4, b4, g4, be4, w5p, b5p):
    def body(aggr, s2r, q2r, g2r, be2r, posr, batr, w3ar, w3pr, b3r, g3r,
             be3r, w4r, b4r, g4r, be4r, w5r, b5r, outr):
        mu2 = s2r[...] / _E
        var2 = jnp.maximum(q2r[...] / _E - mu2 * mu2, 0.0)
        rs2 = lax.rsqrt(var2 + 1e-5)
        aggv = aggr[...]
        a = jnp.where(aggv == -jnp.inf, 0.0,
                      g2r[...] * (aggv - mu2) * rs2 + be2r[...])
        r = jnp.maximum(
            lax.dot_general(a, w3ar[...], (((1,), (1,)), ((), ())),
                            preferred_element_type=jnp.float32)
            + lax.dot_general(posr[...], w3pr[...], (((1,), (1,)), ((), ())),
                              preferred_element_type=jnp.float32)
            + b3r[...], 0.0)
        mu3 = jnp.mean(r, axis=0, keepdims=True)
        var3 = jnp.mean((r - mu3) ** 2, axis=0, keepdims=True)
        q = g3r[...] * (r - mu3) * lax.rsqrt(var3 + 1e-5) + be3r[...]
        bat = batr[...]
        rows = []
        for g in range(_B):
            qm = jnp.where(bat == g, q, -jnp.inf)
            rows.append(jnp.max(qm, axis=0, keepdims=True))
        gf = jnp.concatenate(rows, axis=0)
        gf = jnp.where(gf == -jnp.inf, 0.0, gf)
        h4 = jnp.maximum(
            lax.dot_general(gf, w4r[...], (((1,), (1,)), ((), ())),
                            preferred_element_type=jnp.float32) + b4r[...], 0.0)
        mu4 = jnp.mean(h4, axis=0, keepdims=True)
        var4 = jnp.mean((h4 - mu4) ** 2, axis=0, keepdims=True)
        hn = g4r[...] * (h4 - mu4) * lax.rsqrt(var4 + 1e-5) + be4r[...]
        z = lax.dot_general(hn, w5r[...], (((1,), (1,)), ((), ())),
                            preferred_element_type=jnp.float32) + b5r[...]
        outr[...] = jnp.log1p(jnp.exp(-jnp.abs(z))) + jnp.maximum(z, 0.0)

    return pl.pallas_call(
        body,
        out_shape=jax.ShapeDtypeStruct((_B, 16), jnp.float32),
    )(agg, s2, q2, g2, be2, pos8, batch2d,
      w3a, w3p8, b3, g3, be3, w4, b4, g4, be4, w5p, b5p)


def kernel(x, pos, edge_index, batch, W1, b1, g1, be1, W2, b2, g2, be2,
           W3, b3, g3, be3, W4, b4, g4, be4, W5, b5):
    src = edge_index[0]
    dst = edge_index[1]
    pos8 = jnp.pad(pos, ((0, 0), (0, 5)))
    w1x = W1[:, :128]
    w1p8 = jnp.pad(W1[:, 128:], ((0, 0), (0, 5)))
    w3a = W3[:, :128]
    w3p8 = jnp.pad(W3[:, 128:], ((0, 0), (0, 5)))
    w5p = jnp.pad(W5, ((0, 6), (0, 0)))
    b5p = jnp.pad(b5, (0, 6))

    u, pw = _stage_nodes(x, pos8, w1x, w1p8, b1.reshape(1, 64))
    gu, gp = _sc_gather(src, dst, u, pw)
    h1, s1, q1 = _stage_edge1(gu, gp)
    h2, s2, q2 = _stage_edge2(h1, s1, q1, g1.reshape(1, 64),
                              be1.reshape(1, 64), W2, b2.reshape(1, 128))
    agg = _stage_scatter(dst.reshape(_GRID * 8, _EB // 8), h2)
    out = _stage_head(agg, s2, q2, g2.reshape(1, 128), be2.reshape(1, 128),
                      pos8, batch.reshape(_N, 1),
                      w3a, w3p8, b3.reshape(1, 128), g3.reshape(1, 128),
                      be3.reshape(1, 128), W4, b4.reshape(1, 64),
                      g4.reshape(1, 64), be4.reshape(1, 64), w5p,
                      b5p.reshape(1, 16))
    return out[:, :10]


# speedup vs baseline: 1.6597x; 1.6597x over previous
import functools

import jax
import jax.numpy as jnp
from jax import lax
from jax.experimental import pallas as pl
from jax.experimental.pallas import tpu as pltpu
from jax.experimental.pallas import tpu_sc as plsc

_N = 10000
_E = 320000
_B = 16
_EB = 2000
_GRID = _E // _EB
_CH = 80


def _sc_gather(src, dst, u, pw):
    info = plsc.get_sparse_core_info()
    nc, ns = info.num_cores, info.num_subcores
    nw = nc * ns
    per = _E // nw
    nch = per // _CH

    mesh = plsc.VectorSubcoreMesh(core_axis_name="c", subcore_axis_name="s")

    @functools.partial(
        pl.kernel, mesh=mesh,
        out_type=[jax.ShapeDtypeStruct((_E, 64), jnp.float32),
                  jax.ShapeDtypeStruct((_E, 64), jnp.float32)],
        scratch_types=[pltpu.VMEM((_CH,), jnp.int32),
                       pltpu.VMEM((_CH, 64), jnp.float32),
                       pltpu.SemaphoreType.DMA],
        compiler_params=pltpu.CompilerParams(use_tc_tiling_on_sc=False),
    )
    def k(src_hbm, dst_hbm, u_hbm, pw_hbm, gu_hbm, gp_hbm, idx_v, rows_v, sem):
        wid = lax.axis_index("s") * nc + lax.axis_index("c")
        def body(i, carry):
            base = wid * per + i * _CH
            pltpu.sync_copy(src_hbm.at[pl.ds(base, _CH)], idx_v)
            pltpu.async_copy(u_hbm.at[idx_v], rows_v, sem).wait()
            pltpu.sync_copy(rows_v, gu_hbm.at[pl.ds(base, _CH)])
            pltpu.sync_copy(dst_hbm.at[pl.ds(base, _CH)], idx_v)
            pltpu.async_copy(pw_hbm.at[idx_v], rows_v, sem).wait()
            pltpu.sync_copy(rows_v, gp_hbm.at[pl.ds(base, _CH)])
            return carry
        lax.fori_loop(0, nch, body, 0)

    return k(src, dst, u, pw)


def _stage_nodes(x, pos8, w1x, w1p8, b1):
    def body(xr, pr, wxr, wpr, br, ur, pwr):
        pwv = lax.dot_general(pr[...], wpr[...], (((1,), (1,)), ((), ())),
                              preferred_element_type=jnp.float32)
        xw = lax.dot_general(xr[...], wxr[...], (((1,), (1,)), ((), ())),
                             preferred_element_type=jnp.float32)
        pwr[...] = pwv
        ur[...] = xw + pwv + br[...]

    return pl.pallas_call(
        body,
        out_shape=[jax.ShapeDtypeStruct((_N, 64), jnp.float32),
                   jax.ShapeDtypeStruct((_N, 64), jnp.float32)],
    )(x, pos8, w1x, w1p8, b1)


def _stage_edge1(gu, gp):
    def body(gur, gpr, h1r, s1r, q1r):
        h = jnp.maximum(gur[...] - gpr[...], 0.0)
        h1r[...] = h
        @pl.when(pl.program_id(0) == 0)
        def _():
            s1r[...] = jnp.zeros((1, 64), jnp.float32)
            q1r[...] = jnp.zeros((1, 64), jnp.float32)
        s1r[...] += jnp.sum(h, axis=0, keepdims=True)
        q1r[...] += jnp.sum(h * h, axis=0, keepdims=True)

    return pl.pallas_call(
        body,
        grid=(_GRID,),
        in_specs=[pl.BlockSpec((_EB, 64), lambda i: (i, 0)),
                  pl.BlockSpec((_EB, 64), lambda i: (i, 0))],
        out_specs=[pl.BlockSpec((_EB, 64), lambda i: (i, 0)),
                   pl.BlockSpec((1, 64), lambda i: (0, 0)),
                   pl.BlockSpec((1, 64), lambda i: (0, 0))],
        out_shape=[jax.ShapeDtypeStruct((_E, 64), jnp.float32),
                   jax.ShapeDtypeStruct((1, 64), jnp.float32),
                   jax.ShapeDtypeStruct((1, 64), jnp.float32)],
    )(gu, gp)


def _stage_edge2(h1, s1, q1, g1, be1, w2, b2):
    def body(h1r, s1r, q1r, g1r, be1r, w2r, b2r, h2r, s2r, q2r):
        mu = s1r[...] / _E
        var = jnp.maximum(q1r[...] / _E - mu * mu, 0.0)
        rs = lax.rsqrt(var + 1e-5)
        h1n = g1r[...] * (h1r[...] - mu) * rs + be1r[...]
        h2 = jnp.maximum(
            lax.dot_general(h1n, w2r[...], (((1,), (1,)), ((), ())),
                            preferred_element_type=jnp.float32) + b2r[...], 0.0)
        h2r[...] = h2
        @pl.when(pl.program_id(0) == 0)
        def _():
            s2r[...] = jnp.zeros((1, 128), jnp.float32)
            q2r[...] = jnp.zeros((1, 128), jnp.float32)
        s2r[...] += jnp.sum(h2, axis=0, keepdims=True)
        q2r[...] += jnp.sum(h2 * h2, axis=0, keepdims=True)

    return pl.pallas_call(
        body,
        grid=(_GRID,),
        in_specs=[pl.BlockSpec((_EB, 64), lambda i: (i, 0)),
                  pl.BlockSpec((1, 64), lambda i: (0, 0)),
                  pl.BlockSpec((1, 64), lambda i: (0, 0)),
                  pl.BlockSpec((1, 64), lambda i: (0, 0)),
                  pl.BlockSpec((1, 64), lambda i: (0, 0)),
                  pl.BlockSpec((128, 64), lambda i: (0, 0)),
                  pl.BlockSpec((1, 128), lambda i: (0, 0))],
        out_specs=[pl.BlockSpec((_EB, 128), lambda i: (i, 0)),
                   pl.BlockSpec((1, 128), lambda i: (0, 0)),
                   pl.BlockSpec((1, 128), lambda i: (0, 0))],
        out_shape=[jax.ShapeDtypeStruct((_E, 128), jnp.float32),
                   jax.ShapeDtypeStruct((1, 128), jnp.float32),
                   jax.ShapeDtypeStruct((1, 128), jnp.float32)],
    )(h1, s1, q1, g1, be1, w2, b2)


def _stage_scatter(dst2, h2):
    def body(dstr, h2r, aggr):
        @pl.when(pl.program_id(0) == 0)
        def _():
            aggr[...] = jnp.full((_N, 128), -jnp.inf, jnp.float32)
        def loop(e, carry):
            d = dstr[e // 250, e % 250]
            row = h2r[pl.ds(e, 1), :]
            aggr[pl.ds(d, 1), :] = jnp.maximum(aggr[pl.ds(d, 1), :], row)
            return carry
        lax.fori_loop(0, _EB, loop, 0, unroll=8)

    return pl.pallas_call(
        body,
        grid=(_GRID,),
        in_specs=[pl.BlockSpec((8, 250), lambda i: (i, 0),
                               memory_space=pltpu.SMEM),
                  pl.BlockSpec((_EB, 128), lambda i: (i, 0))],
        out_specs=pl.BlockSpec((_N, 128), lambda i: (0, 0)),
        out_shape=jax.ShapeDtypeStruct((_N, 128), jnp.float32),
    )(dst2, h2)


def _stage_head(agg, s2, q2, g2, be2, pos8, batch2d,
                w3a, w3p8, b3, g3, be3, w4, b4, g4, be4, w5p, b5p):
    def body(aggr, s2r, q2r, g2r, be2r, posr, batr, w3ar, w3pr, b3r, g3r,
             be3r, w4r, b4r, g4r, be4r, w5r, b5r, outr):
        mu2 = s2r[...] / _E
        var2 = jnp.maximum(q2r[...] / _E - mu2 * mu2, 0.0)
        rs2 = lax.rsqrt(var2 + 1e-5)
        aggv = aggr[...]
        a = jnp.where(aggv == -jnp.inf, 0.0,
                      g2r[...] * (aggv - mu2) * rs2 + be2r[...])
        r = jnp.maximum(
            lax.dot_general(a, w3ar[...], (((1,), (1,)), ((), ())),
                            preferred_element_type=jnp.float32)
            + lax.dot_general(posr[...], w3pr[...], (((1,), (1,)), ((), ())),
                              preferred_element_type=jnp.float32)
            + b3r[...], 0.0)
        mu3 = jnp.mean(r, axis=0, keepdims=True)
        var3 = jnp.mean((r - mu3) ** 2, axis=0, keepdims=True)
        q = g3r[...] * (r - mu3) * lax.rsqrt(var3 + 1e-5) + be3r[...]
        bat = batr[...]
        rows = []
        for g in range(_B):
            qm = jnp.where(bat == g, q, -jnp.inf)
            rows.append(jnp.max(qm, axis=0, keepdims=True))
        gf = jnp.concatenate(rows, axis=0)
        gf = jnp.where(gf == -jnp.inf, 0.0, gf)
        h4 = jnp.maximum(
            lax.dot_general(gf, w4r[...], (((1,), (1,)), ((), ())),
                            preferred_element_type=jnp.float32) + b4r[...], 0.0)
        mu4 = jnp.mean(h4, axis=0, keepdims=True)
        var4 = jnp.mean((h4 - mu4) ** 2, axis=0, keepdims=True)
        hn = g4r[...] * (h4 - mu4) * lax.rsqrt(var4 + 1e-5) + be4r[...]
        z = lax.dot_general(hn, w5r[...], (((1,), (1,)), ((), ())),
                            preferred_element_type=jnp.float32) + b5r[...]
        outr[...] = jnp.log1p(jnp.exp(-jnp.abs(z))) + jnp.maximum(z, 0.0)

    return pl.pallas_call(
        body,
        out_shape=jax.ShapeDtypeStruct((_B, 16), jnp.float32),
    )(agg, s2, q2, g2, be2, pos8, batch2d,
      w3a, w3p8, b3, g3, be3, w4, b4, g4, be4, w5p, b5p)


def kernel(x, pos, edge_index, batch, W1, b1, g1, be1, W2, b2, g2, be2,
           W3, b3, g3, be3, W4, b4, g4, be4, W5, b5):
    src = edge_index[0]
    dst = edge_index[1]
    pos8 = jnp.pad(pos, ((0, 0), (0, 5)))
    w1x = W1[:, :128]
    w1p8 = jnp.pad(W1[:, 128:], ((0, 0), (0, 5)))
    w3a = W3[:, :128]
    w3p8 = jnp.pad(W3[:, 128:], ((0, 0), (0, 5)))
    w5p = jnp.pad(W5, ((0, 6), (0, 0)))
    b5p = jnp.pad(b5, (0, 6))

    u, pw = _stage_nodes(x, pos8, w1x, w1p8, b1.reshape(1, 64))
    gu, gp = _sc_gather(src, dst, u, pw)
    h1, s1, q1 = _stage_edge1(gu, gp)
    h2, s2, q2 = _stage_edge2(h1, s1, q1, g1.reshape(1, 64),
                              be1.reshape(1, 64), W2, b2.reshape(1, 128))
    agg = _stage_scatter(dst.reshape(_GRID * 8, _EB // 8), h2)
    out = _stage_head(agg, s2, q2, g2.reshape(1, 128), be2.reshape(1, 128),
                      pos8, batch.reshape(_N, 1),
                      w3a, w3p8, b3.reshape(1, 128), g3.reshape(1, 128),
                      be3.reshape(1, 128), W4, b4.reshape(1, 64),
                      g4.reshape(1, 64), be4.reshape(1, 64), w5p,
                      b5p.reshape(1, 16))
    return out[:, :10]


# speedup vs baseline: 1.6814x; 1.0130x over previous
import functools

import jax
import jax.numpy as jnp
from jax import lax
from jax.experimental import pallas as pl
from jax.experimental.pallas import tpu as pltpu
from jax.experimental.pallas import tpu_sc as plsc

_N = 10000
_E = 320000
_B = 16
_EB = 2000
_GRID = _E // _EB
_CH = 80


def _sc_gather(src, dst, u, pw):
    info = plsc.get_sparse_core_info()
    nc, ns = info.num_cores, info.num_subcores
    nw = nc * ns
    per = _E // nw
    nch = per // _CH

    mesh = plsc.VectorSubcoreMesh(core_axis_name="c", subcore_axis_name="s")

    @functools.partial(
        pl.kernel, mesh=mesh,
        out_type=[jax.ShapeDtypeStruct((_E, 64), jnp.float32),
                  jax.ShapeDtypeStruct((_E, 64), jnp.float32)],
        scratch_types=[pltpu.VMEM((_CH,), jnp.int32),
                       pltpu.VMEM((_CH, 64), jnp.float32),
                       pltpu.SemaphoreType.DMA],
        compiler_params=pltpu.CompilerParams(use_tc_tiling_on_sc=False),
    )
    def k(src_hbm, dst_hbm, u_hbm, pw_hbm, gu_hbm, gp_hbm, idx_v, rows_v, sem):
        wid = lax.axis_index("s") * nc + lax.axis_index("c")
        def body(i, carry):
            base = wid * per + i * _CH
            pltpu.sync_copy(src_hbm.at[pl.ds(base, _CH)], idx_v)
            pltpu.async_copy(u_hbm.at[idx_v], rows_v, sem).wait()
            pltpu.sync_copy(rows_v, gu_hbm.at[pl.ds(base, _CH)])
            pltpu.sync_copy(dst_hbm.at[pl.ds(base, _CH)], idx_v)
            pltpu.async_copy(pw_hbm.at[idx_v], rows_v, sem).wait()
            pltpu.sync_copy(rows_v, gp_hbm.at[pl.ds(base, _CH)])
            return carry
        lax.fori_loop(0, nch, body, 0)

    return k(src, dst, u, pw)


def _stage_nodes(x, pos8, w1x, w1p8, b1):
    def body(xr, pr, wxr, wpr, br, ur, pwr):
        pwv = lax.dot_general(pr[...], wpr[...], (((1,), (1,)), ((), ())),
                              preferred_element_type=jnp.float32)
        xw = lax.dot_general(xr[...], wxr[...], (((1,), (1,)), ((), ())),
                             preferred_element_type=jnp.float32)
        pwr[...] = pwv
        ur[...] = xw + pwv + br[...]

    return pl.pallas_call(
        body,
        out_shape=[jax.ShapeDtypeStruct((_N, 64), jnp.float32),
                   jax.ShapeDtypeStruct((_N, 64), jnp.float32)],
    )(x, pos8, w1x, w1p8, b1)


def _stage_edge1(gu, gp):
    def body(gur, gpr, h1r, s1r, q1r):
        h = jnp.maximum(gur[...] - gpr[...], 0.0)
        h1r[...] = h
        @pl.when(pl.program_id(0) == 0)
        def _():
            s1r[...] = jnp.zeros((1, 64), jnp.float32)
            q1r[...] = jnp.zeros((1, 64), jnp.float32)
        s1r[...] += jnp.sum(h, axis=0, keepdims=True)
        q1r[...] += jnp.sum(h * h, axis=0, keepdims=True)

    return pl.pallas_call(
        body,
        grid=(_GRID,),
        in_specs=[pl.BlockSpec((_EB, 64), lambda i: (i, 0)),
                  pl.BlockSpec((_EB, 64), lambda i: (i, 0))],
        out_specs=[pl.BlockSpec((_EB, 64), lambda i: (i, 0)),
                   pl.BlockSpec((1, 64), lambda i: (0, 0)),
                   pl.BlockSpec((1, 64), lambda i: (0, 0))],
        out_shape=[jax.ShapeDtypeStruct((_E, 64), jnp.float32),
                   jax.ShapeDtypeStruct((1, 64), jnp.float32),
                   jax.ShapeDtypeStruct((1, 64), jnp.float32)],
    )(gu, gp)


def _stage_edge2(h1, s1, q1, g1, be1, w2, b2):
    def body(h1r, s1r, q1r, g1r, be1r, w2r, b2r, h2r, s2r, q2r):
        mu = s1r[...] / _E
        var = jnp.maximum(q1r[...] / _E - mu * mu, 0.0)
        rs = lax.rsqrt(var + 1e-5)
        h1n = g1r[...] * (h1r[...] - mu) * rs + be1r[...]
        h2 = jnp.maximum(
            lax.dot_general(h1n, w2r[...], (((1,), (1,)), ((), ())),
                            preferred_element_type=jnp.float32) + b2r[...], 0.0)
        h2r[...] = h2
        @pl.when(pl.program_id(0) == 0)
        def _():
            s2r[...] = jnp.zeros((1, 128), jnp.float32)
            q2r[...] = jnp.zeros((1, 128), jnp.float32)
        s2r[...] += jnp.sum(h2, axis=0, keepdims=True)
        q2r[...] += jnp.sum(h2 * h2, axis=0, keepdims=True)

    return pl.pallas_call(
        body,
        grid=(_GRID,),
        in_specs=[pl.BlockSpec((_EB, 64), lambda i: (i, 0)),
                  pl.BlockSpec((1, 64), lambda i: (0, 0)),
                  pl.BlockSpec((1, 64), lambda i: (0, 0)),
                  pl.BlockSpec((1, 64), lambda i: (0, 0)),
                  pl.BlockSpec((1, 64), lambda i: (0, 0)),
                  pl.BlockSpec((128, 64), lambda i: (0, 0)),
                  pl.BlockSpec((1, 128), lambda i: (0, 0))],
        out_specs=[pl.BlockSpec((_EB, 128), lambda i: (i, 0)),
                   pl.BlockSpec((1, 128), lambda i: (0, 0)),
                   pl.BlockSpec((1, 128), lambda i: (0, 0))],
        out_shape=[jax.ShapeDtypeStruct((_E, 128), jnp.float32),
                   jax.ShapeDtypeStruct((1, 128), jnp.float32),
                   jax.ShapeDtypeStruct((1, 128), jnp.float32)],
    )(h1, s1, q1, g1, be1, w2, b2)


def _stage_scatter(dst2, h2):
    _K = 8

    def body(dstr, h2r, aggr, *accs):
        @pl.when(pl.program_id(0) == 0)
        def _():
            for a in accs:
                a[...] = jnp.full((_N, 128), -jnp.inf, jnp.float32)
        def loop(i, carry):
            for j in range(_K):
                e = i * _K + j
                d = dstr[e // 250, e % 250]
                row = h2r[pl.ds(e, 1), :]
                a = accs[j]
                a[pl.ds(d, 1), :] = jnp.maximum(a[pl.ds(d, 1), :], row)
            return carry
        lax.fori_loop(0, _EB // _K, loop, 0)
        @pl.when(pl.program_id(0) == _GRID - 1)
        def _():
            m = accs[0][...]
            for a in accs[1:]:
                m = jnp.maximum(m, a[...])
            aggr[...] = m

    return pl.pallas_call(
        body,
        grid=(_GRID,),
        in_specs=[pl.BlockSpec((8, 250), lambda i: (i, 0),
                               memory_space=pltpu.SMEM),
                  pl.BlockSpec((_EB, 128), lambda i: (i, 0))],
        out_specs=pl.BlockSpec((_N, 128), lambda i: (0, 0)),
        out_shape=jax.ShapeDtypeStruct((_N, 128), jnp.float32),
        scratch_shapes=[pltpu.VMEM((_N, 128), jnp.float32)] * _K,
    )(dst2, h2)


def _stage_head(agg, s2, q2, g2, be2, pos8, batch2d,
                w3a, w3p8, b3, g3, be3, w4, b4, g4, be4, w5p, b5p):
    def body(aggr, s2r, q2r, g2r, be2r, posr, batr, w3ar, w3pr, b3r, g3r,
             be3r, w4r, b4r, g4r, be4r, w5r, b5r, outr):
        mu2 = s2r[...] / _E
        var2 = jnp.maximum(q2r[...] / _E - mu2 * mu2, 0.0)
        rs2 = lax.rsqrt(var2 + 1e-5)
        aggv = aggr[...]
        a = jnp.where(aggv == -jnp.inf, 0.0,
                      g2r[...] * (aggv - mu2) * rs2 + be2r[...])
        r = jnp.maximum(
            lax.dot_general(a, w3ar[...], (((1,), (1,)), ((), ())),
                            preferred_element_type=jnp.float32)
            + lax.dot_general(posr[...], w3pr[...], (((1,), (1,)), ((), ())),
                              preferred_element_type=jnp.float32)
            + b3r[...], 0.0)
        mu3 = jnp.mean(r, axis=0, keepdims=True)
        var3 = jnp.mean((r - mu3) ** 2, axis=0, keepdims=True)
        q = g3r[...] * (r - mu3) * lax.rsqrt(var3 + 1e-5) + be3r[...]
        bat = batr[...]
        rows = []
        for g in range(_B):
            qm = jnp.where(bat == g, q, -jnp.inf)
            rows.append(jnp.max(qm, axis=0, keepdims=True))
        gf = jnp.concatenate(rows, axis=0)
        gf = jnp.where(gf == -jnp.inf, 0.0, gf)
        h4 = jnp.maximum(
            lax.dot_general(gf, w4r[...], (((1,), (1,)), ((), ())),
                            preferred_element_type=jnp.float32) + b4r[...], 0.0)
        mu4 = jnp.mean(h4, axis=0, keepdims=True)
        var4 = jnp.mean((h4 - mu4) ** 2, axis=0, keepdims=True)
        hn = g4r[...] * (h4 - mu4) * lax.rsqrt(var4 + 1e-5) + be4r[...]
        z = lax.dot_general(hn, w5r[...], (((1,), (1,)), ((), ())),
                            preferred_element_type=jnp.float32) + b5r[...]
        outr[...] = jnp.log1p(jnp.exp(-jnp.abs(z))) + jnp.maximum(z, 0.0)

    return pl.pallas_call(
        body,
        out_shape=jax.ShapeDtypeStruct((_B, 16), jnp.float32),
    )(agg, s2, q2, g2, be2, pos8, batch2d,
      w3a, w3p8, b3, g3, be3, w4, b4, g4, be4, w5p, b5p)


def kernel(x, pos, edge_index, batch, W1, b1, g1, be1, W2, b2, g2, be2,
           W3, b3, g3, be3, W4, b4, g4, be4, W5, b5):
    src = edge_index[0]
    dst = edge_index[1]
    pos8 = jnp.pad(pos, ((0, 0), (0, 5)))
    w1x = W1[:, :128]
    w1p8 = jnp.pad(W1[:, 128:], ((0, 0), (0, 5)))
    w3a = W3[:, :128]
    w3p8 = jnp.pad(W3[:, 128:], ((0, 0), (0, 5)))
    w5p = jnp.pad(W5, ((0, 6), (0, 0)))
    b5p = jnp.pad(b5, (0, 6))

    u, pw = _stage_nodes(x, pos8, w1x, w1p8, b1.reshape(1, 64))
    gu, gp = _sc_gather(src, dst, u, pw)
    h1, s1, q1 = _stage_edge1(gu, gp)
    h2, s2, q2 = _stage_edge2(h1, s1, q1, g1.reshape(1, 64),
                              be1.reshape(1, 64), W2, b2.reshape(1, 128))
    agg = _stage_scatter(dst.reshape(_GRID * 8, _EB // 8), h2)
    out = _stage_head(agg, s2, q2, g2.reshape(1, 128), be2.reshape(1, 128),
                      pos8, batch.reshape(_N, 1),
                      w3a, w3p8, b3.reshape(1, 128), g3.reshape(1, 128),
                      be3.reshape(1, 128), W4, b4.reshape(1, 64),
                      g4.reshape(1, 64), be4.reshape(1, 64), w5p,
                      b5p.reshape(1, 16))
    return out[:, :10]
